# Initial kernel scaffold; baseline (speedup 1.0000x reference)
import jax
import jax.numpy as jnp
from jax.experimental import pallas as pl


def kernel(t, p, tau):
    raise NotImplementedError("write your pallas kernel here")


# speedup vs baseline: 4.7361x; 4.7361x over previous
import functools

import jax
import jax.numpy as jnp
from jax import lax
from jax.experimental import pallas as pl
from jax.experimental.pallas import tpu as pltpu
from jax.experimental.pallas import tpu_sc as plsc

L = 16
C = 80
NW = 32


def _sc_catmull_rom(t, p, tau):
    N = t.shape[0]
    K, D = p.shape
    n_chunks = N // C
    full_rounds = n_chunks // NW
    rem = n_chunks % NW
    n_groups = C // L

    mesh = plsc.VectorSubcoreMesh(core_axis_name="c", subcore_axis_name="s")

    @functools.partial(
        pl.kernel,
        out_type=jax.ShapeDtypeStruct((N, D), jnp.float32),
        mesh=mesh,
        compiler_params=pltpu.CompilerParams(use_tc_tiling_on_sc=False),
        scratch_types=[
            pltpu.VMEM((C,), jnp.float32),
            pltpu.VMEM((C,), jnp.int32),
            pltpu.VMEM((C,), jnp.int32),
            pltpu.VMEM((C,), jnp.int32),
            pltpu.VMEM((C,), jnp.int32),
            pltpu.VMEM((C,), jnp.float32),
            pltpu.VMEM((C,), jnp.float32),
            pltpu.VMEM((C,), jnp.float32),
            pltpu.VMEM((C,), jnp.float32),
            pltpu.VMEM((C, D), jnp.float32),
            pltpu.VMEM((C, D), jnp.float32),
            pltpu.VMEM((C, D), jnp.float32),
            pltpu.VMEM((C, D), jnp.float32),
            pltpu.VMEM((C, D), jnp.float32),
            pltpu.SemaphoreType.DMA,
        ],
    )
    def run(t_hbm, p_hbm, tau_hbm, out_hbm,
            t_v, i0_v, i1_v, i2_v, i3_v,
            w0_v, w1_v, w2_v, w3_v,
            rp0_v, rp1_v, rm0_v, rm1_v, o_v, sem):
        wid = lax.axis_index("s") * 2 + lax.axis_index("c")

        def process(cid):
            base = cid * C
            pltpu.sync_copy(t_hbm.at[pl.ds(base, C)], t_v)

            @pl.loop(0, n_groups)
            def _(gi):
                sl = pl.ds(gi * L, L)
                tv = t_v[sl]
                s = tv * jnp.float32(K - 1)
                k = jnp.minimum(s.astype(jnp.int32), K - 2)
                u = s - k.astype(jnp.float32)
                u2 = u * u
                u3 = u2 * u
                h00 = 2.0 * u3 - 3.0 * u2 + 1.0
                h10 = u3 - 2.0 * u2 + u
                h01 = -2.0 * u3 + 3.0 * u2
                h11 = u3 - u2
                zero = jnp.zeros_like(u)
                w0_v[sl] = h00
                w1_v[sl] = h01
                w2_v[sl] = jnp.where(k > 0, h10, zero)
                w3_v[sl] = jnp.where(k < K - 2, h11, zero)
                i0_v[sl] = k
                i1_v[sl] = k + 1
                i2_v[sl] = jnp.maximum(k - 1, 0)
                i3_v[sl] = jnp.minimum(k, K - 3)

            c0 = pltpu.async_copy(p_hbm.at[i0_v], rp0_v, sem)
            c1 = pltpu.async_copy(p_hbm.at[i1_v], rp1_v, sem)
            c2 = pltpu.async_copy(tau_hbm.at[i2_v], rm0_v, sem)
            c3 = pltpu.async_copy(tau_hbm.at[i3_v], rm1_v, sem)
            c0.wait()
            c1.wait()
            c2.wait()
            c3.wait()

            @pl.loop(0, n_groups)
            def _(gi):
                sl = pl.ds(gi * L, L)
                w0g = w0_v[sl]
                w1g = w1_v[sl]
                w2g = w2_v[sl]
                w3g = w3_v[sl]
                for c in range(L):
                    row = gi * L + c

                    def splat(vec):
                        return lax.broadcast(
                            lax.squeeze(lax.slice(vec, (c,), (c + 1,)), (0,)),
                            (L,))

                    w0s = splat(w0g)
                    w1s = splat(w1g)
                    w2s = splat(w2g)
                    w3s = splat(w3g)
                    for d in range(D // L):
                        ds_ = pl.ds(d * L, L)
                        o_v[row, ds_] = (w0s * rp0_v[row, ds_]
                                         + w1s * rp1_v[row, ds_]
                                         + w2s * rm0_v[row, ds_]
                                         + w3s * rm1_v[row, ds_])

            pltpu.sync_copy(o_v, out_hbm.at[pl.ds(base, C)])

        n_mine = full_rounds + jnp.where(wid < rem, 1, 0)

        @pl.loop(0, n_mine)
        def _(g):
            process(wid + NW * g)

    return run(t, p, tau)


def kernel(t, p, tau):
    return _sc_catmull_rom(t, p, tau)


# speedup vs baseline: 6.4884x; 1.3700x over previous
import functools

import jax
import jax.numpy as jnp
from jax import lax
from jax.experimental import pallas as pl
from jax.experimental.pallas import tpu as pltpu
from jax.experimental.pallas import tpu_sc as plsc

L = 16
C = 80
NW = 32
NBUF = 2


def _sc_catmull_rom(t, p, tau):
    N = t.shape[0]
    K, D = p.shape
    n_chunks = N // C
    slots = n_chunks // NW + 1
    pairs = (slots + 1) // 2
    n_groups = C // L

    mesh = plsc.VectorSubcoreMesh(core_axis_name="c", subcore_axis_name="s")

    def buf_set():
        return [
            pltpu.VMEM((C,), jnp.float32),
            pltpu.VMEM((C,), jnp.int32),
            pltpu.VMEM((C,), jnp.int32),
            pltpu.VMEM((C,), jnp.int32),
            pltpu.VMEM((C,), jnp.int32),
            pltpu.VMEM((C,), jnp.float32),
            pltpu.VMEM((C,), jnp.float32),
            pltpu.VMEM((C,), jnp.float32),
            pltpu.VMEM((C,), jnp.float32),
            pltpu.VMEM((C, D), jnp.float32),
            pltpu.VMEM((C, D), jnp.float32),
            pltpu.VMEM((C, D), jnp.float32),
            pltpu.VMEM((C, D), jnp.float32),
            pltpu.VMEM((C, D), jnp.float32),
            pltpu.SemaphoreType.DMA,
        ]

    @functools.partial(
        pl.kernel,
        out_type=jax.ShapeDtypeStruct((N, D), jnp.float32),
        mesh=mesh,
        compiler_params=pltpu.CompilerParams(use_tc_tiling_on_sc=False),
        scratch_types=buf_set() + buf_set(),
    )
    def run(t_hbm, p_hbm, tau_hbm, out_hbm, *scratch):
        bufs = (scratch[:15], scratch[15:])
        wid = lax.axis_index("s") * 2 + lax.axis_index("c")

        def prep(g, b):
            (t_v, i0_v, i1_v, i2_v, i3_v,
             w0_v, w1_v, w2_v, w3_v,
             rp0_v, rp1_v, rm0_v, rm1_v, _o_v, sem) = b
            cid = wid + NW * g

            @pl.when(cid < n_chunks)
            def _():
                base = cid * C
                pltpu.sync_copy(t_hbm.at[pl.ds(base, C)], t_v)

                @pl.loop(0, n_groups)
                def _(gi):
                    sl = pl.ds(gi * L, L)
                    tv = t_v[sl]
                    s = tv * jnp.float32(K - 1)
                    k = jnp.minimum(s.astype(jnp.int32), K - 2)
                    u = s - k.astype(jnp.float32)
                    u2 = u * u
                    u3 = u2 * u
                    h00 = 2.0 * u3 - 3.0 * u2 + 1.0
                    h10 = u3 - 2.0 * u2 + u
                    h01 = -2.0 * u3 + 3.0 * u2
                    h11 = u3 - u2
                    zero = jnp.zeros_like(u)
                    w0_v[sl] = h00
                    w1_v[sl] = h01
                    w2_v[sl] = jnp.where(k > 0, h10, zero)
                    w3_v[sl] = jnp.where(k < K - 2, h11, zero)
                    i0_v[sl] = k
                    i1_v[sl] = k + 1
                    i2_v[sl] = jnp.maximum(k - 1, 0)
                    i3_v[sl] = jnp.minimum(k, K - 3)

                pltpu.async_copy(p_hbm.at[i0_v], rp0_v, sem)
                pltpu.async_copy(p_hbm.at[i1_v], rp1_v, sem)
                pltpu.async_copy(tau_hbm.at[i2_v], rm0_v, sem)
                pltpu.async_copy(tau_hbm.at[i3_v], rm1_v, sem)

        def finish(g, b):
            (t_v, i0_v, i1_v, i2_v, i3_v,
             w0_v, w1_v, w2_v, w3_v,
             rp0_v, rp1_v, rm0_v, rm1_v, o_v, sem) = b
            cid = wid + NW * g

            @pl.when(cid < n_chunks)
            def _():
                base = cid * C
                pltpu.make_async_copy(p_hbm.at[i0_v], rp0_v, sem).wait()
                pltpu.make_async_copy(p_hbm.at[i1_v], rp1_v, sem).wait()
                pltpu.make_async_copy(tau_hbm.at[i2_v], rm0_v, sem).wait()
                pltpu.make_async_copy(tau_hbm.at[i3_v], rm1_v, sem).wait()

                @pl.loop(0, n_groups)
                def _(gi):
                    sl = pl.ds(gi * L, L)
                    w0g = w0_v[sl]
                    w1g = w1_v[sl]
                    w2g = w2_v[sl]
                    w3g = w3_v[sl]
                    for c in range(L):
                        row = gi * L + c

                        def splat(vec):
                            return lax.broadcast(
                                lax.squeeze(
                                    lax.slice(vec, (c,), (c + 1,)), (0,)),
                                (L,))

                        w0s = splat(w0g)
                        w1s = splat(w1g)
                        w2s = splat(w2g)
                        w3s = splat(w3g)
                        for d in range(D // L):
                            ds_ = pl.ds(d * L, L)
                            o_v[row, ds_] = (w0s * rp0_v[row, ds_]
                                             + w1s * rp1_v[row, ds_]
                                             + w2s * rm0_v[row, ds_]
                                             + w3s * rm1_v[row, ds_])

                pltpu.sync_copy(o_v, out_hbm.at[pl.ds(base, C)])

        prep(0, bufs[0])

        @pl.loop(0, pairs)
        def _(j):
            g0 = 2 * j
            prep(g0 + 1, bufs[1])
            finish(g0, bufs[0])
            prep(g0 + 2, bufs[0])
            finish(g0 + 1, bufs[1])

    return run(t, p, tau)


def kernel(t, p, tau):
    return _sc_catmull_rom(t, p, tau)


# speedup vs baseline: 6.9556x; 1.0720x over previous
import functools

import jax
import jax.numpy as jnp
from jax import lax
from jax.experimental import pallas as pl
from jax.experimental.pallas import tpu as pltpu
from jax.experimental.pallas import tpu_sc as plsc

L = 16
C = 160
H = 80
NW = 32
NBUF = 2


def _sc_catmull_rom(t, p, tau):
    N = t.shape[0]
    K, D = p.shape
    n_chunks = N // C
    slots = n_chunks // NW + 1
    pairs = (slots + 1) // 2
    n_groups = C // L

    mesh = plsc.VectorSubcoreMesh(core_axis_name="c", subcore_axis_name="s")

    def buf_set():
        return [
            pltpu.VMEM((C,), jnp.float32),
            pltpu.VMEM((C,), jnp.int32),
            pltpu.VMEM((C,), jnp.int32),
            pltpu.VMEM((C,), jnp.int32),
            pltpu.VMEM((C,), jnp.int32),
            pltpu.VMEM((C,), jnp.float32),
            pltpu.VMEM((C,), jnp.float32),
            pltpu.VMEM((C,), jnp.float32),
            pltpu.VMEM((C,), jnp.float32),
            pltpu.VMEM((C, D), jnp.float32),
            pltpu.VMEM((C, D), jnp.float32),
            pltpu.VMEM((C, D), jnp.float32),
            pltpu.VMEM((C, D), jnp.float32),
            pltpu.VMEM((C, D), jnp.float32),
            pltpu.SemaphoreType.DMA,
        ]

    @functools.partial(
        pl.kernel,
        out_type=jax.ShapeDtypeStruct((N, D), jnp.float32),
        mesh=mesh,
        compiler_params=pltpu.CompilerParams(use_tc_tiling_on_sc=False),
        scratch_types=buf_set() + buf_set(),
    )
    def run(t_hbm, p_hbm, tau_hbm, out_hbm, *scratch):
        bufs = (scratch[:15], scratch[15:])
        wid = lax.axis_index("s") * 2 + lax.axis_index("c")

        def prep(g, b):
            (t_v, i0_v, i1_v, i2_v, i3_v,
             w0_v, w1_v, w2_v, w3_v,
             rp0_v, rp1_v, rm0_v, rm1_v, _o_v, sem) = b
            cid = wid + NW * g

            @pl.when(cid < n_chunks)
            def _():
                base = cid * C
                pltpu.sync_copy(t_hbm.at[pl.ds(base, C)], t_v)

                @pl.loop(0, n_groups)
                def _(gi):
                    sl = pl.ds(gi * L, L)
                    tv = t_v[sl]
                    s = tv * jnp.float32(K - 1)
                    k = jnp.minimum(s.astype(jnp.int32), K - 2)
                    u = s - k.astype(jnp.float32)
                    u2 = u * u
                    u3 = u2 * u
                    h00 = 2.0 * u3 - 3.0 * u2 + 1.0
                    h10 = u3 - 2.0 * u2 + u
                    h01 = -2.0 * u3 + 3.0 * u2
                    h11 = u3 - u2
                    zero = jnp.zeros_like(u)
                    w0_v[sl] = h00
                    w1_v[sl] = h01
                    w2_v[sl] = jnp.where(k > 0, h10, zero)
                    w3_v[sl] = jnp.where(k < K - 2, h11, zero)
                    i0_v[sl] = k
                    i1_v[sl] = k + 1
                    i2_v[sl] = jnp.maximum(k - 1, 0)
                    i3_v[sl] = jnp.minimum(k, K - 3)

                for h in range(C // H):
                    hs = pl.ds(h * H, H)
                    pltpu.async_copy(p_hbm.at[i0_v.at[hs]], rp0_v.at[hs], sem)
                    pltpu.async_copy(p_hbm.at[i1_v.at[hs]], rp1_v.at[hs], sem)
                    pltpu.async_copy(tau_hbm.at[i2_v.at[hs]], rm0_v.at[hs], sem)
                    pltpu.async_copy(tau_hbm.at[i3_v.at[hs]], rm1_v.at[hs], sem)

        def finish(g, b):
            (t_v, i0_v, i1_v, i2_v, i3_v,
             w0_v, w1_v, w2_v, w3_v,
             rp0_v, rp1_v, rm0_v, rm1_v, o_v, sem) = b
            cid = wid + NW * g

            @pl.when(cid < n_chunks)
            def _():
                base = cid * C
                for h in range(C // H):
                    hs = pl.ds(h * H, H)
                    pltpu.make_async_copy(
                        p_hbm.at[i0_v.at[hs]], rp0_v.at[hs], sem).wait()
                    pltpu.make_async_copy(
                        p_hbm.at[i1_v.at[hs]], rp1_v.at[hs], sem).wait()
                    pltpu.make_async_copy(
                        tau_hbm.at[i2_v.at[hs]], rm0_v.at[hs], sem).wait()
                    pltpu.make_async_copy(
                        tau_hbm.at[i3_v.at[hs]], rm1_v.at[hs], sem).wait()

                @pl.loop(0, n_groups)
                def _(gi):
                    sl = pl.ds(gi * L, L)
                    w0g = w0_v[sl]
                    w1g = w1_v[sl]
                    w2g = w2_v[sl]
                    w3g = w3_v[sl]
                    for c in range(L):
                        row = gi * L + c

                        def splat(vec):
                            return lax.broadcast(
                                lax.squeeze(
                                    lax.slice(vec, (c,), (c + 1,)), (0,)),
                                (L,))

                        w0s = splat(w0g)
                        w1s = splat(w1g)
                        w2s = splat(w2g)
                        w3s = splat(w3g)
                        for d in range(D // L):
                            ds_ = pl.ds(d * L, L)
                            o_v[row, ds_] = (w0s * rp0_v[row, ds_]
                                             + w1s * rp1_v[row, ds_]
                                             + w2s * rm0_v[row, ds_]
                                             + w3s * rm1_v[row, ds_])

                pltpu.sync_copy(o_v, out_hbm.at[pl.ds(base, C)])

        prep(0, bufs[0])

        @pl.loop(0, pairs)
        def _(j):
            g0 = 2 * j
            prep(g0 + 1, bufs[1])
            finish(g0, bufs[0])
            prep(g0 + 2, bufs[0])
            finish(g0 + 1, bufs[1])

    return run(t, p, tau)


def kernel(t, p, tau):
    return _sc_catmull_rom(t, p, tau)
